# baseline (speedup 1.0000x reference)
import functools

import jax
import jax.numpy as jnp
from jax import lax
from jax.experimental import pallas as pl
from jax.experimental.pallas import tpu as pltpu
from jax.experimental.pallas import tpu_sc as plsc

HIDDEN = 256
B, L, NUM_SPANS = 4, 512, 256
ZPAD = 8
LP = L + ZPAD
NLANE = 16
NCHUNK = HIDDEN // NLANE

_NC, _NS = 2, 16
BPC = B // _NC
ROWS = L // _NS
SPC = NUM_SPANS * BPC
SPW = SPC // _NS
SPB = _NS // BPC
NW_TOTAL = _NC * _NS


def _sc_body(h_hbm, se_hbm, e_out_hbm, s_out_hbm, p_hbm, tot_hbm,
             hbuf, pbuf, zbuf, totbuf, idx_v, erows_v, srows_v, sem):
    cid = lax.axis_index("c")
    sid = lax.axis_index("s")

    for bl in range(BPC):
        batch = cid * BPC + bl
        pltpu.sync_copy(h_hbm.at[batch, pl.ds(sid * ROWS, ROWS), :], hbuf)

        def row_step(r, accs):
            new = []
            for j in range(NCHUNK):
                a = accs[j] + hbuf[r, pl.ds(j * NLANE, NLANE)]
                pbuf[r, pl.ds(j * NLANE, NLANE)] = a
                new.append(a)
            return tuple(new)

        zero = jnp.zeros((NLANE,), jnp.float32)
        lax.fori_loop(0, ROWS, row_step, (zero,) * NCHUNK)
        pltpu.sync_copy(pbuf.at[pl.ds(ROWS - 8, 8), :],
                        tot_hbm.at[pl.ds((batch * _NS + sid) * 8, 8)])
        plsc.subcore_barrier()

        pltpu.sync_copy(tot_hbm.at[pl.ds(batch * _NS * 8, _NS * 8)], totbuf)

        def carry_step(t, accs):
            return tuple(accs[j] + totbuf[t * 8 + 7, pl.ds(j * NLANE, NLANE)]
                         for j in range(NCHUNK))

        carry = lax.fori_loop(0, sid, carry_step, (zero,) * NCHUNK)

        def fix_step(r, _):
            for j in range(NCHUNK):
                pbuf[r, pl.ds(j * NLANE, NLANE)] = (
                    pbuf[r, pl.ds(j * NLANE, NLANE)] + carry[j]
                )
            return 0

        lax.fori_loop(0, ROWS, fix_step, 0)
        pltpu.sync_copy(pbuf, p_hbm.at[pl.ds(batch * LP + ZPAD + sid * ROWS, ROWS)])

        @pl.when(sid == 0)
        def _():
            def zrow_step(r, _):
                for j in range(NCHUNK):
                    zbuf[r, pl.ds(j * NLANE, NLANE)] = (
                        hbuf[r, pl.ds(j * NLANE, NLANE)]
                        - hbuf[r, pl.ds(j * NLANE, NLANE)]
                    )
                return 0

            lax.fori_loop(0, ZPAD, zrow_step, 0)
            pltpu.sync_copy(zbuf, p_hbm.at[pl.ds(batch * LP, ZPAD)])

    plsc.subcore_barrier()

    base = cid * SPC + sid * SPW
    reg = (cid * BPC + sid // SPB) * LP

    wrk = cid * _NS + sid
    pltpu.sync_copy(se_hbm.at[pl.ds(wrk * 2 * SPW, 2 * SPW)], idx_v)
    for c in range(SPW // NLANE):
        ev = idx_v[pl.ds(c * NLANE, NLANE)]
        sv = idx_v[pl.ds(SPW + c * NLANE, NLANE)]
        idx_v[pl.ds(c * NLANE, NLANE)] = ev + (reg + ZPAD)
        idx_v[pl.ds(SPW + c * NLANE, NLANE)] = sv + (reg + ZPAD - 1)

    ecopy = pltpu.async_copy(p_hbm.at[idx_v.at[pl.ds(0, SPW)]], erows_v, sem)
    ecopy.wait()
    scopy = pltpu.async_copy(p_hbm.at[idx_v.at[pl.ds(SPW, SPW)]], srows_v, sem)
    scopy.wait()
    pltpu.sync_copy(erows_v, e_out_hbm.at[pl.ds(base, SPW)])
    pltpu.sync_copy(srows_v, s_out_hbm.at[pl.ds(base, SPW)])


def _mlp_kernel(e_ref, s_ref, len_ref, w1_ref, b1_ref, w2_ref, b2_ref, out_ref):
    reps = (e_ref[...] - s_ref[...]) * (1.0 / len_ref[...].astype(jnp.float32))
    x = lax.dot_general(
        reps, w1_ref[...], (((1,), (0,)), ((), ())),
        precision=lax.Precision.DEFAULT,
        preferred_element_type=jnp.float32,
    )
    x = jnp.maximum(x + b1_ref[...], 0.0)
    out = lax.dot_general(
        x, w2_ref[...], (((1,), (0,)), ((), ())),
        precision=lax.Precision.DEFAULT,
        preferred_element_type=jnp.float32,
    )
    out_ref[...] = out + b2_ref[...]


def kernel(h, span_idx, W1, b1, W2, b2):
    span_idx = span_idx.astype(jnp.int32)
    starts = span_idx[:, :, 0].reshape(B * NUM_SPANS)
    ends = span_idx[:, :, 1].reshape(B * NUM_SPANS)
    se = jnp.concatenate(
        [ends.reshape(NW_TOTAL, SPW), starts.reshape(NW_TOTAL, SPW)], axis=1
    ).reshape(NW_TOTAL * 2 * SPW)

    sc_reduce = functools.partial(
        pl.kernel,
        mesh=plsc.VectorSubcoreMesh(core_axis_name="c", subcore_axis_name="s"),
        out_type=(
            jax.ShapeDtypeStruct((B * NUM_SPANS, HIDDEN), jnp.float32),
            jax.ShapeDtypeStruct((B * NUM_SPANS, HIDDEN), jnp.float32),
            jax.ShapeDtypeStruct((B * LP, HIDDEN), jnp.float32),
            jax.ShapeDtypeStruct((B * _NS * 8, HIDDEN), jnp.float32),
        ),
        scratch_types=[
            pltpu.VMEM((ROWS, HIDDEN), jnp.float32),
            pltpu.VMEM((ROWS, HIDDEN), jnp.float32),
            pltpu.VMEM((ZPAD, HIDDEN), jnp.float32),
            pltpu.VMEM((_NS * 8, HIDDEN), jnp.float32),
            pltpu.VMEM((2 * SPW,), jnp.int32),
            pltpu.VMEM((SPW, HIDDEN), jnp.float32),
            pltpu.VMEM((SPW, HIDDEN), jnp.float32),
            pltpu.SemaphoreType.DMA,
        ],
    )(_sc_body)
    e_rows, s_rows, _, _ = sc_reduce(h, se)

    lengths = (ends - starts + 1).reshape(B * NUM_SPANS, 1)
    out = pl.pallas_call(
        _mlp_kernel,
        in_specs=[
            pl.BlockSpec((B * NUM_SPANS, HIDDEN), lambda: (0, 0)),
            pl.BlockSpec((B * NUM_SPANS, HIDDEN), lambda: (0, 0)),
            pl.BlockSpec((B * NUM_SPANS, 1), lambda: (0, 0)),
            pl.BlockSpec((HIDDEN, 4 * HIDDEN), lambda: (0, 0)),
            pl.BlockSpec((1, 4 * HIDDEN), lambda: (0, 0)),
            pl.BlockSpec((4 * HIDDEN, HIDDEN), lambda: (0, 0)),
            pl.BlockSpec((1, HIDDEN), lambda: (0, 0)),
        ],
        out_specs=pl.BlockSpec((B * NUM_SPANS, HIDDEN), lambda: (0, 0)),
        out_shape=jax.ShapeDtypeStruct((B * NUM_SPANS, HIDDEN), jnp.float32),
    )(e_rows, s_rows, lengths, W1, b1.reshape(1, 4 * HIDDEN), W2, b2.reshape(1, HIDDEN))
    return out.reshape(B, NUM_SPANS, HIDDEN)


# speedup vs baseline: 1.3959x; 1.3959x over previous
import functools

import jax
import jax.numpy as jnp
from jax import lax
from jax.experimental import pallas as pl
from jax.experimental.pallas import tpu as pltpu
from jax.experimental.pallas import tpu_sc as plsc

HIDDEN = 256
B, L, NUM_SPANS = 4, 512, 256
LP = L + 8
NLANE = 16

_NC, _NS = 2, 16
NW = _NC * _NS
SPW = (B * NUM_SPANS) // NW
WPB = NUM_SPANS // SPW


def _prefix_kernel(h_ref, p_ref):
    hb = h_ref[0]
    row = lax.broadcasted_iota(jnp.int32, (LP, L), 0)
    col = lax.broadcasted_iota(jnp.int32, (LP, L), 1)
    tri = (col < row).astype(jnp.bfloat16)
    hi = hb.astype(jnp.bfloat16)
    lo = (hb - hi.astype(jnp.float32)).astype(jnp.bfloat16)
    dn = (((1,), (0,)), ((), ()))
    p_ref[0] = (
        lax.dot_general(tri, hi, dn, preferred_element_type=jnp.float32)
        + lax.dot_general(tri, lo, dn, preferred_element_type=jnp.float32)
    )


def _sc_body(p_hbm, se_hbm, e_out_hbm, s_out_hbm, idx_v, erows_v, srows_v, sem):
    wid = lax.axis_index("s") * _NC + lax.axis_index("c")
    base = wid * SPW
    row_off = (wid // WPB) * LP

    pltpu.sync_copy(se_hbm.at[pl.ds(wid * 2 * SPW, 2 * SPW)], idx_v)
    for c in range(SPW // NLANE):
        ev = idx_v[pl.ds(c * NLANE, NLANE)]
        sv = idx_v[pl.ds(SPW + c * NLANE, NLANE)]
        idx_v[pl.ds(c * NLANE, NLANE)] = ev + (row_off + 1)
        idx_v[pl.ds(SPW + c * NLANE, NLANE)] = sv + row_off

    ecopy = pltpu.async_copy(p_hbm.at[idx_v.at[pl.ds(0, SPW)]], erows_v, sem)
    scopy = pltpu.async_copy(p_hbm.at[idx_v.at[pl.ds(SPW, SPW)]], srows_v, sem)
    ecopy.wait()
    scopy.wait()
    pltpu.sync_copy(erows_v, e_out_hbm.at[pl.ds(base, SPW)])
    pltpu.sync_copy(srows_v, s_out_hbm.at[pl.ds(base, SPW)])


def _mlp_kernel(e_ref, s_ref, len_ref, w1_ref, b1_ref, w2_ref, b2_ref, out_ref):
    reps = (e_ref[...] - s_ref[...]) * (1.0 / len_ref[...].astype(jnp.float32))
    x = lax.dot_general(
        reps, w1_ref[...], (((1,), (0,)), ((), ())),
        precision=lax.Precision.DEFAULT,
        preferred_element_type=jnp.float32,
    )
    x = jnp.maximum(x + b1_ref[...], 0.0)
    out = lax.dot_general(
        x, w2_ref[...], (((1,), (0,)), ((), ())),
        precision=lax.Precision.DEFAULT,
        preferred_element_type=jnp.float32,
    )
    out_ref[...] = out + b2_ref[...]


def kernel(h, span_idx, W1, b1, W2, b2):
    span_idx = span_idx.astype(jnp.int32)
    starts = span_idx[:, :, 0].reshape(B * NUM_SPANS)
    ends = span_idx[:, :, 1].reshape(B * NUM_SPANS)
    se = jnp.concatenate(
        [ends.reshape(NW, SPW), starts.reshape(NW, SPW)], axis=1
    ).reshape(NW * 2 * SPW)

    prefix = pl.pallas_call(
        _prefix_kernel,
        grid=(B,),
        in_specs=[pl.BlockSpec((1, L, HIDDEN), lambda b: (b, 0, 0))],
        out_specs=pl.BlockSpec((1, LP, HIDDEN), lambda b: (b, 0, 0)),
        out_shape=jax.ShapeDtypeStruct((B, LP, HIDDEN), jnp.float32),
    )(h)
    p_flat = prefix.reshape(B * LP, HIDDEN)

    sc_gather = functools.partial(
        pl.kernel,
        mesh=plsc.VectorSubcoreMesh(core_axis_name="c", subcore_axis_name="s"),
        out_type=(
            jax.ShapeDtypeStruct((B * NUM_SPANS, HIDDEN), jnp.float32),
            jax.ShapeDtypeStruct((B * NUM_SPANS, HIDDEN), jnp.float32),
        ),
        scratch_types=[
            pltpu.VMEM((2 * SPW,), jnp.int32),
            pltpu.VMEM((SPW, HIDDEN), jnp.float32),
            pltpu.VMEM((SPW, HIDDEN), jnp.float32),
            pltpu.SemaphoreType.DMA,
        ],
    )(_sc_body)
    e_rows, s_rows = sc_gather(p_flat, se)

    lengths = (ends - starts + 1).reshape(B * NUM_SPANS, 1)
    out = pl.pallas_call(
        _mlp_kernel,
        in_specs=[
            pl.BlockSpec((B * NUM_SPANS, HIDDEN), lambda: (0, 0)),
            pl.BlockSpec((B * NUM_SPANS, HIDDEN), lambda: (0, 0)),
            pl.BlockSpec((B * NUM_SPANS, 1), lambda: (0, 0)),
            pl.BlockSpec((HIDDEN, 4 * HIDDEN), lambda: (0, 0)),
            pl.BlockSpec((1, 4 * HIDDEN), lambda: (0, 0)),
            pl.BlockSpec((4 * HIDDEN, HIDDEN), lambda: (0, 0)),
            pl.BlockSpec((1, HIDDEN), lambda: (0, 0)),
        ],
        out_specs=pl.BlockSpec((B * NUM_SPANS, HIDDEN), lambda: (0, 0)),
        out_shape=jax.ShapeDtypeStruct((B * NUM_SPANS, HIDDEN), jnp.float32),
    )(e_rows, s_rows, lengths, W1, b1.reshape(1, 4 * HIDDEN), W2, b2.reshape(1, HIDDEN))
    return out.reshape(B, NUM_SPANS, HIDDEN)
